# baseline (speedup 1.0000x reference)
import functools

import jax
import jax.numpy as jnp
from jax import lax
from jax.experimental import pallas as pl
from jax.experimental.pallas import tpu as pltpu
from jax.experimental.pallas import tpu_sc as plsc

NC = 2
NS = 16
EB = 128


K = 8


def _make_segment_sum(n, d, nacc, nb):
    zps = nacc // NS
    nch = nb // K

    mesh = plsc.VectorSubcoreMesh(
        core_axis_name="c", subcore_axis_name="s",
        num_cores=NC, num_subcores=NS)

    @functools.partial(
        pl.kernel,
        out_type=jax.ShapeDtypeStruct((NC, nacc, d), jnp.float32),
        mesh=mesh,
        scratch_types=[
            pltpu.VMEM_SHARED((nacc, d), jnp.float32),
            pltpu.VMEM((K, EB), jnp.int32),
            pltpu.VMEM((K, EB), jnp.int32),
            pltpu.VMEM((K, EB), jnp.int32),
            pltpu.VMEM((K, EB), jnp.int32),
            pltpu.VMEM((K, EB), jnp.int32),
            pltpu.VMEM((K, EB), jnp.int32),
            pltpu.VMEM((EB, d), jnp.float32),
            pltpu.VMEM((EB, d), jnp.float32),
            pltpu.SemaphoreType.DMA,
            pltpu.SemaphoreType.DMA,
            pltpu.SemaphoreType.DMA,
            pltpu.SemaphoreType.DMA,
        ],
    )
    def seg_sum(h_hbm, src_hbm, dst_hbm, zero_hbm, out_hbm,
                acc, sb0, sb1, sb2, db0, db1, db2, rows0, rows1,
                g0, g1, a0, a1):
        c = lax.axis_index("c")
        s = lax.axis_index("s")
        wid = c * NS + s
        row0 = wid * nb

        pltpu.sync_copy(zero_hbm, acc.at[pl.ds(s * zps, zps)])
        plsc.subcore_barrier()

        sbufs = (sb0, sb1, sb2)
        dbufs = (db0, db1, db2)
        rbufs = (rows0, rows1)
        gsems = (g0, g1)
        asems = (a0, a1)

        def load_idx(ch):
            pltpu.sync_copy(src_hbm.at[pl.ds(row0 + ch * K, K)], sbufs[ch % 3])
            pltpu.sync_copy(dst_hbm.at[pl.ds(row0 + ch * K, K)], dbufs[ch % 3])

        def gather_desc(jj):
            ch, pos = divmod(jj, K)
            return pltpu.make_async_copy(
                h_hbm.at[sbufs[ch % 3].at[pos]], rbufs[jj % 2], gsems[jj % 2])

        def scatter_desc(jj):
            ch, pos = divmod(jj, K)
            return pltpu.make_async_copy(
                rbufs[jj % 2], acc.at[dbufs[ch % 3].at[pos]], asems[jj % 2])

        load_idx(0)
        for jj in range(nb):
            ch, pos = divmod(jj, K)
            if pos == 0 and ch + 1 < nch:
                load_idx(ch + 1)
            if jj >= 2:
                scatter_desc(jj - 2).wait()
            gather_desc(jj).start()
            if jj >= 1:
                gather_desc(jj - 1).wait()
                scatter_desc(jj - 1).start(add=True)
        gather_desc(nb - 1).wait()
        scatter_desc(nb - 1).start(add=True)
        scatter_desc(nb - 2).wait()
        scatter_desc(nb - 1).wait()
        plsc.subcore_barrier()

        pltpu.sync_copy(acc.at[pl.ds(s * zps, zps)],
                        out_hbm.at[c].at[pl.ds(s * zps, zps)])

    return seg_sum


def _linear_relu(parts, w, b, n, d, blk):
    nb = n // blk

    def body(p0_ref, p1_ref, w_ref, b_ref, o_ref):
        msgs = p0_ref[0] + p1_ref[0]
        y = lax.dot_general(msgs, w_ref[...], (((1,), (0,)), ((), ())),
                            preferred_element_type=jnp.float32)
        o_ref[...] = jnp.maximum(y + b_ref[...], 0.0)

    return pl.pallas_call(
        body,
        grid=(nb,),
        in_specs=[
            pl.BlockSpec((1, blk, d), lambda i: (0, i, 0)),
            pl.BlockSpec((1, blk, d), lambda i: (1, i, 0)),
            pl.BlockSpec((d, d), lambda i: (0, 0)),
            pl.BlockSpec((1, d), lambda i: (0, 0)),
        ],
        out_specs=pl.BlockSpec((blk, d), lambda i: (i, 0)),
        out_shape=jax.ShapeDtypeStruct((n, d), jnp.float32),
    )(parts, parts, w, b.reshape(1, d))


def kernel(x, edge_index, W1, b1, W2, b2):
    n, d = x.shape
    e = edge_index.shape[1]

    loop = jnp.arange(n, dtype=jnp.int32)
    src = jnp.concatenate([edge_index[0].astype(jnp.int32), loop])
    dst = jnp.concatenate([edge_index[1].astype(jnp.int32), loop])

    etot = e + n
    nw = NC * NS
    nb = -(-etot // (nw * EB * K)) * K
    epad = nw * nb * EB - etot
    src = jnp.concatenate([src, jnp.zeros((epad,), jnp.int32)])
    dst = jnp.concatenate([dst, jnp.full((epad,), n, jnp.int32)])
    src = src.reshape(nw * nb, EB)
    dst = dst.reshape(nw * nb, EB)

    nacc = -(-(n + 1) // (8 * NS)) * (8 * NS)
    zeros = jnp.zeros((nacc // NS, d), jnp.float32)

    seg = _make_segment_sum(n, d, nacc, nb)

    parts1 = seg(x, src, dst, zeros)
    h1 = _linear_relu(parts1, W1, b1, n, d, blk=1000)
    parts2 = seg(h1, src, dst, zeros)
    h2 = _linear_relu(parts2, W2, b2, n, d, blk=1000)
    return h2
